# Initial kernel scaffold; baseline (speedup 1.0000x reference)
import jax
import jax.numpy as jnp
from jax.experimental import pallas as pl


def kernel(edge_index, features, args, lin_W, lin_b, gcn0_W, gcn0_b, gcn1_W, gcn1_b, lin1_W, lin1_b):
    raise NotImplementedError("write your pallas kernel here")


# speedup vs baseline: 2.4141x; 2.4141x over previous
import functools

import jax
import jax.numpy as jnp
from jax.experimental import pallas as pl
from jax.experimental.pallas import tpu as pltpu

T = 2
NUM_CLASSES = 128
NUM_HE = 128
F = 128
N = 10000
E = 320000
N3 = 3 * N
SCALE = F ** (-0.5)


def _epilogue_a_body(af2_ref, lin1W_ref, lin1b_ref, Hs_ref, hf_ref):
    af2 = af2_ref[...]
    allv = jax.lax.dot_general(
        jax.nn.relu(af2), lin1W_ref[...], (((1,), (0,)), ((), ())),
        preferred_element_type=jnp.float32) + lin1b_ref[...]
    iota = jax.lax.broadcasted_iota(jnp.int32, (N, NUM_HE), 1)
    H = jnp.zeros((N, NUM_HE), jnp.float32)
    for i in range(T + 1):
        blk = allv[i * N:(i + 1) * N]
        m = jnp.max(blk, axis=1, keepdims=True)
        first = jnp.min(jnp.where(blk == m, iota, NUM_HE), axis=1, keepdims=True)
        H = H + jnp.where(iota == first, 1.0, 0.0)
    mask = jnp.where(H > 0, 1.0, 0.0)
    hf = jax.lax.dot_general(mask, af2[:N], (((0,), (0,)), ((), ())),
                             preferred_element_type=jnp.float32)
    e = jnp.exp(H - jnp.max(H, axis=0, keepdims=True))
    Hs_ref[...] = e / jnp.sum(e, axis=0, keepdims=True)
    hf_ref[...] = hf


def _epilogue_a(af2, lin1_W, lin1_b):
    return pl.pallas_call(
        _epilogue_a_body,
        out_shape=(jax.ShapeDtypeStruct((N, NUM_HE), jnp.float32),
                   jax.ShapeDtypeStruct((NUM_HE, F), jnp.float32)),
    )(af2, lin1_W, lin1_b.reshape(1, NUM_HE))


def _dots_body(af_ref, hf_ref, out_ref):
    out_ref[...] = jax.lax.dot_general(
        af_ref[...], hf_ref[...], (((1,), (1,)), ((), ())),
        preferred_element_type=jnp.float32) * SCALE


def _dots(all_features, hf):
    blk = 6000
    return pl.pallas_call(
        _dots_body,
        grid=(N3 // blk,),
        in_specs=[pl.BlockSpec((blk, F), lambda i: (i, 0)),
                  pl.BlockSpec((NUM_HE, F), lambda i: (0, 0))],
        out_specs=pl.BlockSpec((blk, NUM_HE), lambda i: (i, 0)),
        out_shape=jax.ShapeDtypeStruct((N3, NUM_HE), jnp.float32),
    )(all_features, hf)


def kernel(edge_index, features, args, lin_W, lin_b, gcn0_W, gcn0_b, gcn1_W,
           gcn1_b, lin1_W, lin1_b):
    src = edge_index[0]
    dst = edge_index[1]
    T0 = features @ lin_W[0] + lin_b[0]
    T1 = features @ lin_W[1] + lin_b[1]

    j = (2 * jnp.arange(E)) % E
    d0 = dst[j]
    d1 = dst[j + 1]
    a = features[src]
    nb = jnp.maximum(jnp.linalg.norm(T0, axis=1), 1e-8)
    s0 = jnp.sum(a * T0[d0], axis=1) / nb[d0]
    s1 = jnp.sum(a * T0[d1], axis=1) / nb[d1]
    best_repl = dst + jnp.where(s1 > s0, N, 0).astype(jnp.int32)

    cnt = jnp.zeros((N3,), jnp.float32).at[dst].add(1.0).at[best_repl].add(1.0)
    deg = cnt + 1.0 + (jnp.arange(N3) < 2 * N).astype(jnp.float32)
    dinv = jax.lax.rsqrt(deg)

    all_features = jnp.concatenate([features, T0, T1], axis=0)
    x = jax.nn.relu(all_features)

    def conv(x, W, b):
        hs = (x @ W) * dinv[:, None]
        rows = hs[src]
        acc = jnp.zeros((N3, F), jnp.float32).at[dst].add(rows).at[best_repl].add(rows)
        out = acc * dinv[:, None] + hs * dinv[:, None]
        out = out.at[:N].add(hs[:N] * dinv[:N, None])
        out = out.at[N:2 * N].add(hs[:N] * dinv[N:2 * N, None])
        return out + b

    x = jax.nn.relu(conv(x, gcn0_W, gcn0_b))
    af2 = conv(x, gcn1_W, gcn1_b)

    Hs, hf = _epilogue_a(af2, lin1_W, lin1_b)
    dots = _dots(all_features, hf)
    return (Hs, hf, dots)


# speedup vs baseline: 10.3527x; 4.2885x over previous
import functools

import jax
import jax.numpy as jnp
from jax import lax
from jax.experimental import pallas as pl
from jax.experimental.pallas import tpu as pltpu
from jax.experimental.pallas import tpu_sc as plsc

T = 2
NUM_CLASSES = 128
NUM_HE = 128
F = 128
FH = 64
N = 10000
E = 320000
N3 = 3 * N
N2 = 2 * N
SCALE = F ** (-0.5)
NCORE = 2
NSUB = 16
NW = NCORE * NSUB
ER = E // 128

_mesh = plsc.VectorSubcoreMesh(core_axis_name="c", subcore_axis_name="s")


_PW = E // NW
_SK = 80
_SNCH = _PW // _SK


def _select_body(f_hbm, t0n_hbm, src_hbm, dst_hbm, repl_hbm,
                 srcbuf, djbuf, dobuf, replbuf, ar0, br0, ar1, br1,
                 sem0, sem1):
    c = lax.axis_index("c")
    s = lax.axis_index("s")
    w = c * NSUB + s
    pb = w * _PW
    jb = (w % NSUB) * (2 * _PW)
    pltpu.sync_copy(src_hbm.at[pl.ds(pb, _PW)], srcbuf)
    pltpu.sync_copy(dst_hbm.at[pl.ds(jb, 2 * _PW)], djbuf)
    pltpu.sync_copy(dst_hbm.at[pl.ds(pb, _PW)], dobuf)

    def fire(ci, ar, br, sem):
        off = ci * _SK
        pltpu.async_copy(f_hbm.at[srcbuf.at[pl.ds(off, _SK)]], ar, sem)
        pltpu.async_copy(t0n_hbm.at[djbuf.at[pl.ds(2 * off, _SK)]],
                         br.at[pl.ds(0, _SK)], sem)
        pltpu.async_copy(t0n_hbm.at[djbuf.at[pl.ds(2 * off + _SK, _SK)]],
                         br.at[pl.ds(_SK, _SK)], sem)

    def waitg(ci, ar, br, sem):
        off = ci * _SK
        pltpu.make_async_copy(f_hbm.at[srcbuf.at[pl.ds(off, _SK)]], ar, sem).wait()
        pltpu.make_async_copy(t0n_hbm.at[djbuf.at[pl.ds(2 * off, _SK)]],
                              br.at[pl.ds(0, _SK)], sem).wait()
        pltpu.make_async_copy(t0n_hbm.at[djbuf.at[pl.ds(2 * off + _SK, _SK)]],
                              br.at[pl.ds(_SK, _SK)], sem).wait()

    lane = lax.iota(jnp.int32, 16)
    nvec = jnp.broadcast_to(jnp.int32(N), (16,))
    zvec = jnp.broadcast_to(jnp.int32(0), (16,))

    def compute(ci, ar, br):
        @plsc.parallel_loop(0, _SK // 16, step=1)
        def _(g):
            dvv = jnp.zeros((16,), jnp.float32)
            for q in range(16):
                p = 16 * g + q
                acc0 = jnp.zeros((16,), jnp.float32)
                acc1 = jnp.zeros((16,), jnp.float32)
                for cb in range(8):
                    va = ar[p, pl.ds(cb * 16, 16)]
                    acc0 = acc0 + va * br[2 * p, pl.ds(cb * 16, 16)]
                    acc1 = acc1 + va * br[2 * p + 1, pl.ds(cb * 16, 16)]
                diff = acc1 - acc0
                s = diff[0]
                for l in range(1, 16):
                    s = s + diff[l]
                dvv = jnp.where(lane == q, jnp.broadcast_to(s, (16,)), dvv)
            dv = dobuf[pl.ds(ci * _SK + 16 * g, 16)]
            replbuf[pl.ds(ci * _SK + 16 * g, 16)] = (
                dv + jnp.where(dvv > 0.0, nvec, zvec))

    fire(0, ar0, br0, sem0)

    def step(j, carry):
        c0 = 2 * j
        waitg(c0, ar0, br0, sem0)
        fire(c0 + 1, ar1, br1, sem1)
        compute(c0, ar0, br0)
        waitg(c0 + 1, ar1, br1, sem1)
        fire(c0 + 2, ar0, br0, sem0)
        compute(c0 + 1, ar1, br1)
        return carry

    lax.fori_loop(0, (_SNCH - 1) // 2, step, 0)
    waitg(_SNCH - 1, ar0, br0, sem0)
    compute(_SNCH - 1, ar0, br0)
    pltpu.sync_copy(replbuf, repl_hbm.at[pl.ds(pb, _PW)])


_select = pl.kernel(
    _select_body,
    out_type=jax.ShapeDtypeStruct((E,), jnp.int32),
    mesh=_mesh,
    scratch_types=[
        pltpu.VMEM((_PW,), jnp.int32),
        pltpu.VMEM((2 * _PW,), jnp.int32),
        pltpu.VMEM((_PW,), jnp.int32),
        pltpu.VMEM((_PW,), jnp.int32),
        pltpu.VMEM((_SK, F), jnp.float32),
        pltpu.VMEM((2 * _SK, F), jnp.float32),
        pltpu.VMEM((_SK, F), jnp.float32),
        pltpu.VMEM((2 * _SK, F), jnp.float32),
        pltpu.SemaphoreType.DMA,
        pltpu.SemaphoreType.DMA,
    ],
)


def _spread(ibuf, buf2d, pad):
    padv = jnp.broadcast_to(pad, (16,)).astype(jnp.int32)

    def rr(r, carry):
        for cb in range(8):
            buf2d[r, pl.ds(cb * 16, 16)] = ibuf[pl.ds(128 * r + cb * 16, 16)]
        return carry

    lax.fori_loop(0, 78, rr, 0)
    buf2d[78, pl.ds(0, 16)] = ibuf[pl.ds(9984, 16)]
    for cb in range(1, 8):
        buf2d[78, pl.ds(cb * 16, 16)] = padv
    for cb in range(8):
        buf2d[79, pl.ds(cb * 16, 16)] = padv


def _hist_body(dst_hbm, repl_hbm, out_hbm, hist, ibuf, dbuf, rbuf, ones, zbuf,
               sem):
    c = lax.axis_index("c")
    s = lax.axis_index("s")
    base = c * (E // 2) + s * 10000
    trash = 20000 + s

    def fill(i, carry):
        ones[i, :] = jnp.ones((16,), jnp.float32)
        zbuf[i, :] = jnp.zeros((16,), jnp.float32)
        return carry
    lax.fori_loop(0, 128, fill, 0)

    pltpu.sync_copy(dst_hbm.at[pl.ds(base, 10000)], ibuf)
    _spread(ibuf, dbuf, trash)
    pltpu.sync_copy(repl_hbm.at[pl.ds(base, 10000)], ibuf)
    _spread(ibuf, rbuf, trash)

    for q in range(9):
        pltpu.sync_copy(zbuf, hist.at[pl.ds(s * 1248 + q * 128, 128)])
    pltpu.sync_copy(zbuf.at[pl.ds(0, 96)], hist.at[pl.ds(s * 1248 + 1152, 96)])

    @pl.when(s == 15)
    def _():
        pltpu.sync_copy(zbuf.at[pl.ds(0, 32)], hist.at[pl.ds(19968, 32)])
        pltpu.sync_copy(zbuf.at[pl.ds(0, 16)], hist.at[pl.ds(20000, 16)])
    plsc.subcore_barrier()

    prev = None
    for row in range(80):
        d1 = pltpu.async_copy(ones, hist.at[dbuf.at[row]], sem, add=True)
        d2 = pltpu.async_copy(ones, hist.at[rbuf.at[row]], sem, add=True)
        if prev is not None:
            prev[0].wait()
            prev[1].wait()
        prev = (d1, d2)
    prev[0].wait()
    prev[1].wait()
    plsc.subcore_barrier()

    pltpu.sync_copy(hist.at[pl.ds(s * 1248, 1248)],
                    out_hbm.at[c].at[pl.ds(s * 1248, 1248)])

    @pl.when(s == 15)
    def _():
        pltpu.sync_copy(hist.at[pl.ds(19968, 32)],
                        out_hbm.at[c].at[pl.ds(19968, 32)])


_hist = pl.kernel(
    _hist_body,
    out_type=jax.ShapeDtypeStruct((2, 20000, 16), jnp.float32),
    mesh=_mesh,
    scratch_types=[
        pltpu.VMEM_SHARED((20016, 16), jnp.float32),
        pltpu.VMEM((10000,), jnp.int32),
        pltpu.VMEM((80, 128), jnp.int32),
        pltpu.VMEM((80, 128), jnp.int32),
        pltpu.VMEM((128, 16), jnp.float32),
        pltpu.VMEM((128, 16), jnp.float32),
        pltpu.SemaphoreType.DMA,
    ],
)


def _agg_body(hs_hbm, src_hbm, dst_hbm, repl_hbm, out_hbm,
              acc, sidx, didx, ridx, rowA, rowB, gsA, gsB, ssA, ssB):
    c = lax.axis_index("c")
    s = lax.axis_index("s")
    trashv = jnp.broadcast_to(10000 + s, (16,)).astype(jnp.int32)
    cv = jnp.broadcast_to(c, (16,)).astype(jnp.int32)

    def to_local(v):
        q = (v * 26844) >> 27
        local = (q >> 1) * 5000 + (v - q * 5000)
        return jnp.where((q & 1) == cv, local, trashv)

    def zfill(i, carry):
        for cb in range(8):
            rowA[i, pl.ds(cb * 16, 16)] = jnp.zeros((16,), jnp.float32)
        return carry
    lax.fori_loop(0, 128, zfill, 0)
    for q in range(4):
        pltpu.sync_copy(rowA, acc.at[pl.ds(s * 624 + q * 128, 128)])
    pltpu.sync_copy(rowA.at[pl.ds(0, 112)], acc.at[pl.ds(s * 624 + 512, 112)])

    @pl.when(s == 15)
    def _():
        pltpu.sync_copy(rowA.at[pl.ds(0, 32)], acc.at[pl.ds(9984, 32)])
    plsc.subcore_barrier()

    def fire_g(r, row, sem):
        return pltpu.async_copy(hs_hbm.at[sidx.at[r]], row, sem)

    def fire_s(r, row, sem):
        return (pltpu.async_copy(row, acc.at[didx.at[r]], sem, add=True),
                pltpu.async_copy(row, acc.at[ridx.at[r]], sem, add=True))

    def blk_body(b, carry):
        row0 = s * 160 + b * 8
        pltpu.sync_copy(src_hbm.at[pl.ds(row0, 8)], sidx)
        pltpu.sync_copy(dst_hbm.at[pl.ds(row0, 8)], didx)
        pltpu.sync_copy(repl_hbm.at[pl.ds(row0, 8)], ridx)
        for r in range(8):
            for cb in range(8):
                sl = pl.ds(cb * 16, 16)
                didx[r, sl] = to_local(didx[r, sl])
                ridx[r, sl] = to_local(ridx[r, sl])
        gA = fire_g(0, rowA, gsA)
        for r in range(0, 8, 2):
            gA.wait()
            sA = fire_s(r, rowA, ssA)
            gB = fire_g(r + 1, rowB, gsB)
            gB.wait()
            sA[0].wait()
            sA[1].wait()
            sB = fire_s(r + 1, rowB, ssB)
            if r + 2 < 8:
                gA = fire_g(r + 2, rowA, gsA)
            sB[0].wait()
            sB[1].wait()
        return carry

    lax.fori_loop(0, 20, blk_body, 0)
    plsc.subcore_barrier()
    pltpu.sync_copy(acc.at[pl.ds(s * 624, 624)],
                    out_hbm.at[c].at[pl.ds(s * 624, 624)])

    @pl.when(s == 15)
    def _():
        pltpu.sync_copy(acc.at[pl.ds(9984, 16)],
                        out_hbm.at[c].at[pl.ds(9984, 16)])


_agg = pl.kernel(
    _agg_body,
    out_type=jax.ShapeDtypeStruct((2, 10000, F), jnp.float32),
    mesh=_mesh,
    scratch_types=[
        pltpu.VMEM_SHARED((10016, F), jnp.float32),
        pltpu.VMEM((8, 128), jnp.int32),
        pltpu.VMEM((8, 128), jnp.int32),
        pltpu.VMEM((8, 128), jnp.int32),
        pltpu.VMEM((128, F), jnp.float32),
        pltpu.VMEM((128, F), jnp.float32),
        pltpu.SemaphoreType.DMA,
        pltpu.SemaphoreType.DMA,
        pltpu.SemaphoreType.DMA,
        pltpu.SemaphoreType.DMA,
    ],
)


def _prologue_body(f_ref, W_ref, b_ref, t0_ref, t1_ref, t0n_ref):
    f = f_ref[...]
    t0 = jax.lax.dot_general(f, W_ref[0], (((1,), (0,)), ((), ())),
                             preferred_element_type=jnp.float32) + b_ref[0, 0]
    t1 = jax.lax.dot_general(f, W_ref[1], (((1,), (0,)), ((), ())),
                             preferred_element_type=jnp.float32) + b_ref[1, 0]
    ss = jnp.sum(t0 * t0, axis=1, keepdims=True)
    nb = jnp.maximum(jnp.sqrt(ss), 1e-8)
    t0_ref[...] = t0
    t1_ref[...] = t1
    t0n_ref[...] = t0 / nb


def _prologue(features, lin_W, lin_b):
    blk = BLK
    return pl.pallas_call(
        _prologue_body,
        grid=(N // blk,),
        in_specs=[pl.BlockSpec((blk, F), lambda i: (i, 0)),
                  pl.BlockSpec((T, F, F), lambda i: (0, 0, 0)),
                  pl.BlockSpec((T, 1, F), lambda i: (0, 0, 0))],
        out_specs=[pl.BlockSpec((blk, F), lambda i: (i, 0))] * 3,
        out_shape=[jax.ShapeDtypeStruct((N, F), jnp.float32)] * 3,
    )(features, lin_W, lin_b.reshape(T, 1, F))


BLK = 2000


def _dinv_blk(hp, i):
    cnt = hp[0, :, 0:1] + hp[1, :, 0:1]
    base = jnp.where(i < 10, 2.0, 1.0)
    return jax.lax.rsqrt(cnt + base)


def _layer_body(first, acc_ref, h_ref, hsh_ref, hp_ref, b_ref, W_ref,
                out_ref):
    i = pl.program_id(0)
    dinv = _dinv_blk(hp_ref[...], i)
    if first:
        x = jax.nn.relu(acc_ref[...])
    else:
        ex_same = jnp.where(i < 5, 1.0, 0.0)
        ex_shift = jnp.where((i >= 5) & (i < 10), 1.0, 0.0)
        pre = ((acc_ref[...] + h_ref[...] + ex_same * h_ref[...]
                + ex_shift * hsh_ref[...]) * dinv + b_ref[...])
        x = jax.nn.relu(pre)
    out_ref[...] = jax.lax.dot_general(
        x, W_ref[...], (((1,), (0,)), ((), ())),
        preferred_element_type=jnp.float32) * dinv


def _layer0(af, hpx, gcn0_W):
    blk = BLK
    dummy = jnp.zeros((8, 128), jnp.float32)
    return pl.pallas_call(
        functools.partial(_layer_body, True),
        grid=(N3 // blk,),
        in_specs=[pl.BlockSpec((blk, F), lambda i: (i, 0)),
                  pl.BlockSpec((8, 128), lambda i: (0, 0)),
                  pl.BlockSpec((8, 128), lambda i: (0, 0)),
                  pl.BlockSpec((2, blk, 16), lambda i: (0, i, 0)),
                  pl.BlockSpec((8, 128), lambda i: (0, 0)),
                  pl.BlockSpec((F, F), lambda i: (0, 0))],
        out_specs=pl.BlockSpec((blk, F), lambda i: (i, 0)),
        out_shape=jax.ShapeDtypeStruct((N3, F), jnp.float32),
    )(af, dummy, dummy, hpx, dummy, gcn0_W)


def _layer_mid(accx, h, hpx, b, W):
    blk = BLK
    return pl.pallas_call(
        functools.partial(_layer_body, False),
        grid=(N3 // blk,),
        in_specs=[pl.BlockSpec((blk, F), lambda i: (i, 0)),
                  pl.BlockSpec((blk, F), lambda i: (i, 0)),
                  pl.BlockSpec((blk, F),
                               lambda i: (jnp.clip(i - 5, 0, N3 // blk - 1), 0)),
                  pl.BlockSpec((2, blk, 16), lambda i: (0, i, 0)),
                  pl.BlockSpec((1, F), lambda i: (0, 0)),
                  pl.BlockSpec((F, F), lambda i: (0, 0))],
        out_specs=pl.BlockSpec((blk, F), lambda i: (i, 0)),
        out_shape=jax.ShapeDtypeStruct((N3, F), jnp.float32),
    )(accx, h, h, hpx, b.reshape(1, F), W)


def _final_body(acc_ref, h_ref, hsh_ref, hp_ref, b_ref, out_ref):
    i = pl.program_id(0)
    dinv = _dinv_blk(hp_ref[...], i)
    ex_same = jnp.where(i < 5, 1.0, 0.0)
    ex_shift = jnp.where((i >= 5) & (i < 10), 1.0, 0.0)
    out_ref[...] = ((acc_ref[...] + h_ref[...] + ex_same * h_ref[...]
                     + ex_shift * hsh_ref[...]) * dinv + b_ref[...])


def _layer_final(accx, h, hpx, b):
    blk = BLK
    return pl.pallas_call(
        _final_body,
        grid=(N3 // blk,),
        in_specs=[pl.BlockSpec((blk, F), lambda i: (i, 0)),
                  pl.BlockSpec((blk, F), lambda i: (i, 0)),
                  pl.BlockSpec((blk, F),
                               lambda i: (jnp.clip(i - 5, 0, N3 // blk - 1), 0)),
                  pl.BlockSpec((2, blk, 16), lambda i: (0, i, 0)),
                  pl.BlockSpec((1, F), lambda i: (0, 0))],
        out_specs=pl.BlockSpec((blk, F), lambda i: (i, 0)),
        out_shape=jax.ShapeDtypeStruct((N3, F), jnp.float32),
    )(accx, h, h, hpx, b.reshape(1, F))


def _epilogue_a_body(af2_ref, lin1W_ref, lin1b_ref, Hs_ref, hf_ref):
    af2 = af2_ref[...]
    allv = jax.lax.dot_general(
        jax.nn.relu(af2), lin1W_ref[...], (((1,), (0,)), ((), ())),
        preferred_element_type=jnp.float32) + lin1b_ref[...]
    iota = jax.lax.broadcasted_iota(jnp.int32, (N, NUM_HE), 1)
    H = jnp.zeros((N, NUM_HE), jnp.float32)
    for i in range(T + 1):
        blk = allv[i * N:(i + 1) * N]
        m = jnp.max(blk, axis=1, keepdims=True)
        first = jnp.min(jnp.where(blk == m, iota, NUM_HE), axis=1, keepdims=True)
        H = H + jnp.where(iota == first, 1.0, 0.0)
    mask = jnp.where(H > 0, 1.0, 0.0)
    hf = jax.lax.dot_general(mask, af2[:N], (((0,), (0,)), ((), ())),
                             preferred_element_type=jnp.float32)
    e = jnp.exp(H - jnp.max(H, axis=0, keepdims=True))
    Hs_ref[...] = e / jnp.sum(e, axis=0, keepdims=True)
    hf_ref[...] = hf


def _epilogue_a(af2, lin1_W, lin1_b):
    return pl.pallas_call(
        _epilogue_a_body,
        out_shape=(jax.ShapeDtypeStruct((N, NUM_HE), jnp.float32),
                   jax.ShapeDtypeStruct((NUM_HE, F), jnp.float32)),
    )(af2, lin1_W, lin1_b.reshape(1, NUM_HE))


def _dots_body(af_ref, hf_ref, out_ref):
    out_ref[...] = jax.lax.dot_general(
        af_ref[...], hf_ref[...], (((1,), (1,)), ((), ())),
        preferred_element_type=jnp.float32) * SCALE


def _dots(all_features, hf):
    blk = 6000
    return pl.pallas_call(
        _dots_body,
        grid=(N3 // blk,),
        in_specs=[pl.BlockSpec((blk, F), lambda i: (i, 0)),
                  pl.BlockSpec((NUM_HE, F), lambda i: (0, 0))],
        out_specs=pl.BlockSpec((blk, NUM_HE), lambda i: (i, 0)),
        out_shape=jax.ShapeDtypeStruct((N3, NUM_HE), jnp.float32),
    )(all_features, hf)


def _unshuffle(part):
    return jnp.concatenate(
        [part[0, :5000], part[1, :5000], part[0, 5000:], part[1, 5000:],
         jnp.zeros((N, F), jnp.float32)], axis=0)


_SC_SELECT = True
_SC_HIST = False
_SC_AGG = True


def _select_xla(features, t0n, src, dst):
    j = (2 * jnp.arange(E)) % E
    a = features[src]
    s0 = jnp.sum(a * t0n[dst[j]], axis=1)
    s1 = jnp.sum(a * t0n[dst[j + 1]], axis=1)
    return dst + jnp.where(s1 > s0, N, 0).astype(jnp.int32)


def _hist_xla(dst, repl):
    cnt0 = jnp.zeros((20000,), jnp.float32).at[dst].add(1.0)
    cnt1 = jnp.zeros((20000,), jnp.float32).at[repl].add(1.0)
    return jnp.stack([jnp.tile(cnt0[:, None], (1, 16)),
                      jnp.tile(cnt1[:, None], (1, 16))])


def _agg_xla(hs, src, dst, repl):
    rows = hs[src]
    accf = (jnp.zeros((20000, F), jnp.float32).at[dst].add(rows)
            .at[repl].add(rows))
    loc = jnp.concatenate([accf[:5000], accf[10000:15000],
                           accf[5000:10000], accf[15000:]], axis=0)
    return loc.reshape(2, 10000, F)


def kernel(edge_index, features, args, lin_W, lin_b, gcn0_W, gcn0_b, gcn1_W,
           gcn1_b, lin1_W, lin1_b):
    src = edge_index[0]
    dst = edge_index[1]

    T0, T1, t0n = _prologue(features, lin_W, lin_b)
    all_features = jnp.concatenate([features, T0, T1], axis=0)

    if _SC_SELECT:
        repl = _select(features, t0n, src, dst)
    else:
        repl = _select_xla(features, t0n, src, dst)

    hpart = _hist(dst, repl) if _SC_HIST else _hist_xla(dst, repl)
    hpx = jnp.concatenate(
        [hpart, jnp.zeros((2, N, 16), jnp.float32)], axis=1)

    padn = 2560 * 128 - E
    src_p = jnp.concatenate([src, jnp.zeros((padn,), jnp.int32)]).reshape(2560, 128)
    dst_p = jnp.concatenate([dst, jnp.full((padn,), 20000, jnp.int32)]).reshape(2560, 128)
    repl_p = jnp.concatenate([repl, jnp.full((padn,), 20000, jnp.int32)]).reshape(2560, 128)

    h0s = _layer0(all_features, hpx, gcn0_W)
    part0 = (_agg(h0s, src_p, dst_p, repl_p) if _SC_AGG
             else _agg_xla(h0s, src, dst, repl))

    h1s = _layer_mid(_unshuffle(part0), h0s, hpx, gcn0_b, gcn1_W)
    part1 = (_agg(h1s, src_p, dst_p, repl_p) if _SC_AGG
             else _agg_xla(h1s, src, dst, repl))

    af2 = _layer_final(_unshuffle(part1), h1s, hpx, gcn1_b)

    Hs, hf = _epilogue_a(af2, lin1_W, lin1_b)
    dots = _dots(all_features, hf)
    return (Hs, hf, dots)


# speedup vs baseline: 11.1877x; 1.0807x over previous
import functools

import jax
import jax.numpy as jnp
from jax import lax
from jax.experimental import pallas as pl
from jax.experimental.pallas import tpu as pltpu
from jax.experimental.pallas import tpu_sc as plsc

T = 2
NUM_CLASSES = 128
NUM_HE = 128
F = 128
FH = 64
N = 10000
E = 320000
N3 = 3 * N
N2 = 2 * N
SCALE = F ** (-0.5)
NCORE = 2
NSUB = 16
NW = NCORE * NSUB
ER = E // 128

_mesh = plsc.VectorSubcoreMesh(core_axis_name="c", subcore_axis_name="s")


_PW = E // NW
_SK = 80
_SNCH = _PW // _SK


def _select_body(f_hbm, t0n_hbm, src_hbm, dst_hbm, repl_hbm,
                 srcbuf, djbuf, dobuf, replbuf, ar0, br0, ar1, br1,
                 sem0, sem1):
    c = lax.axis_index("c")
    s = lax.axis_index("s")
    w = c * NSUB + s
    pb = w * _PW
    jb = (w % NSUB) * (2 * _PW)
    pltpu.sync_copy(src_hbm.at[pl.ds(pb, _PW)], srcbuf)
    pltpu.sync_copy(dst_hbm.at[pl.ds(jb, 2 * _PW)], djbuf)
    pltpu.sync_copy(dst_hbm.at[pl.ds(pb, _PW)], dobuf)

    def fire(ci, ar, br, sem):
        off = ci * _SK
        pltpu.async_copy(f_hbm.at[srcbuf.at[pl.ds(off, _SK)]], ar, sem)
        pltpu.async_copy(t0n_hbm.at[djbuf.at[pl.ds(2 * off, _SK)]],
                         br.at[pl.ds(0, _SK)], sem)
        pltpu.async_copy(t0n_hbm.at[djbuf.at[pl.ds(2 * off + _SK, _SK)]],
                         br.at[pl.ds(_SK, _SK)], sem)

    def waitg(ci, ar, br, sem):
        off = ci * _SK
        pltpu.make_async_copy(f_hbm.at[srcbuf.at[pl.ds(off, _SK)]], ar, sem).wait()
        pltpu.make_async_copy(t0n_hbm.at[djbuf.at[pl.ds(2 * off, _SK)]],
                              br.at[pl.ds(0, _SK)], sem).wait()
        pltpu.make_async_copy(t0n_hbm.at[djbuf.at[pl.ds(2 * off + _SK, _SK)]],
                              br.at[pl.ds(_SK, _SK)], sem).wait()

    lane = lax.iota(jnp.int32, 16)
    nvec = jnp.broadcast_to(jnp.int32(N), (16,))
    zvec = jnp.broadcast_to(jnp.int32(0), (16,))

    def compute(ci, ar, br):
        @plsc.parallel_loop(0, _SK // 16, step=1)
        def _(g):
            dvv = jnp.zeros((16,), jnp.float32)
            for q in range(16):
                p = 16 * g + q
                acc0 = jnp.zeros((16,), jnp.float32)
                acc1 = jnp.zeros((16,), jnp.float32)
                for cb in range(8):
                    va = ar[p, pl.ds(cb * 16, 16)]
                    acc0 = acc0 + va * br[2 * p, pl.ds(cb * 16, 16)]
                    acc1 = acc1 + va * br[2 * p + 1, pl.ds(cb * 16, 16)]
                diff = acc1 - acc0
                s = diff[0]
                for l in range(1, 16):
                    s = s + diff[l]
                dvv = jnp.where(lane == q, jnp.broadcast_to(s, (16,)), dvv)
            dv = dobuf[pl.ds(ci * _SK + 16 * g, 16)]
            replbuf[pl.ds(ci * _SK + 16 * g, 16)] = (
                dv + jnp.where(dvv > 0.0, nvec, zvec))

    fire(0, ar0, br0, sem0)

    def step(j, carry):
        c0 = 2 * j
        waitg(c0, ar0, br0, sem0)
        fire(c0 + 1, ar1, br1, sem1)
        compute(c0, ar0, br0)
        waitg(c0 + 1, ar1, br1, sem1)
        fire(c0 + 2, ar0, br0, sem0)
        compute(c0 + 1, ar1, br1)
        return carry

    lax.fori_loop(0, (_SNCH - 1) // 2, step, 0)
    waitg(_SNCH - 1, ar0, br0, sem0)
    compute(_SNCH - 1, ar0, br0)
    pltpu.sync_copy(replbuf, repl_hbm.at[pl.ds(pb, _PW)])


_select = pl.kernel(
    _select_body,
    out_type=jax.ShapeDtypeStruct((E,), jnp.int32),
    mesh=_mesh,
    scratch_types=[
        pltpu.VMEM((_PW,), jnp.int32),
        pltpu.VMEM((2 * _PW,), jnp.int32),
        pltpu.VMEM((_PW,), jnp.int32),
        pltpu.VMEM((_PW,), jnp.int32),
        pltpu.VMEM((_SK, F), jnp.float32),
        pltpu.VMEM((2 * _SK, F), jnp.float32),
        pltpu.VMEM((_SK, F), jnp.float32),
        pltpu.VMEM((2 * _SK, F), jnp.float32),
        pltpu.SemaphoreType.DMA,
        pltpu.SemaphoreType.DMA,
    ],
)


def _hist_body(dst_hbm, repl_hbm, out_hbm, hist, didx, ridx, ones, sem):
    c = lax.axis_index("c")
    s = lax.axis_index("s")
    trashv = jnp.broadcast_to(10000 + s, (16,)).astype(jnp.int32)
    cv = jnp.broadcast_to(c, (16,)).astype(jnp.int32)

    def to_local(v):
        q = (v * 26844) >> 27
        local = (q >> 1) * 5000 + (v - q * 5000)
        return jnp.where((q & 1) == cv, local, trashv)

    def zfill(i, carry):
        for cb in range(8):
            ones[i, pl.ds(cb * 16, 16)] = jnp.zeros((16,), jnp.float32)
        return carry
    lax.fori_loop(0, 128, zfill, 0)
    for q in range(4):
        pltpu.sync_copy(ones, hist.at[pl.ds(s * 624 + q * 128, 128)])
    pltpu.sync_copy(ones.at[pl.ds(0, 112)], hist.at[pl.ds(s * 624 + 512, 112)])

    @pl.when(s == 15)
    def _():
        pltpu.sync_copy(ones.at[pl.ds(0, 32)], hist.at[pl.ds(9984, 32)])

    def ofill(i, carry):
        for cb in range(8):
            ones[i, pl.ds(cb * 16, 16)] = jnp.ones((16,), jnp.float32)
        return carry
    lax.fori_loop(0, 128, ofill, 0)
    plsc.subcore_barrier()

    def blk_body(b, carry):
        row0 = s * 160 + b * 8
        pltpu.sync_copy(dst_hbm.at[pl.ds(row0, 8)], didx)
        pltpu.sync_copy(repl_hbm.at[pl.ds(row0, 8)], ridx)
        for r in range(8):
            for cb in range(8):
                sl = pl.ds(cb * 16, 16)
                didx[r, sl] = to_local(didx[r, sl])
                ridx[r, sl] = to_local(ridx[r, sl])
        prev = None
        for r in range(8):
            d1 = pltpu.async_copy(ones, hist.at[didx.at[r]], sem, add=True)
            d2 = pltpu.async_copy(ones, hist.at[ridx.at[r]], sem, add=True)
            if prev is not None:
                prev[0].wait()
                prev[1].wait()
            prev = (d1, d2)
        prev[0].wait()
        prev[1].wait()
        return carry

    lax.fori_loop(0, 20, blk_body, 0)
    plsc.subcore_barrier()
    pltpu.sync_copy(hist.at[pl.ds(s * 624, 624)],
                    out_hbm.at[c].at[pl.ds(s * 624, 624)])

    @pl.when(s == 15)
    def _():
        pltpu.sync_copy(hist.at[pl.ds(9984, 16)],
                        out_hbm.at[c].at[pl.ds(9984, 16)])


_hist = pl.kernel(
    _hist_body,
    out_type=jax.ShapeDtypeStruct((2, 10000, F), jnp.float32),
    mesh=_mesh,
    scratch_types=[
        pltpu.VMEM_SHARED((10016, F), jnp.float32),
        pltpu.VMEM((8, 128), jnp.int32),
        pltpu.VMEM((8, 128), jnp.int32),
        pltpu.VMEM((128, F), jnp.float32),
        pltpu.SemaphoreType.DMA,
    ],
)


def _agg_body(hs_hbm, src_hbm, dst_hbm, repl_hbm, out_hbm,
              acc, sidx, didx, ridx, rowA, rowB, gsA, gsB, ssA, ssB):
    c = lax.axis_index("c")
    s = lax.axis_index("s")
    trashv = jnp.broadcast_to(10000 + s, (16,)).astype(jnp.int32)
    cv = jnp.broadcast_to(c, (16,)).astype(jnp.int32)

    def to_local(v):
        q = (v * 26844) >> 27
        local = (q >> 1) * 5000 + (v - q * 5000)
        return jnp.where((q & 1) == cv, local, trashv)

    def zfill(i, carry):
        for cb in range(8):
            rowA[i, pl.ds(cb * 16, 16)] = jnp.zeros((16,), jnp.float32)
        return carry
    lax.fori_loop(0, 128, zfill, 0)
    for q in range(4):
        pltpu.sync_copy(rowA, acc.at[pl.ds(s * 624 + q * 128, 128)])
    pltpu.sync_copy(rowA.at[pl.ds(0, 112)], acc.at[pl.ds(s * 624 + 512, 112)])

    @pl.when(s == 15)
    def _():
        pltpu.sync_copy(rowA.at[pl.ds(0, 32)], acc.at[pl.ds(9984, 32)])
    plsc.subcore_barrier()

    def fire_g(r, row, sem):
        return pltpu.async_copy(hs_hbm.at[sidx.at[r]], row, sem)

    def fire_s(r, row, sem):
        return (pltpu.async_copy(row, acc.at[didx.at[r]], sem, add=True),
                pltpu.async_copy(row, acc.at[ridx.at[r]], sem, add=True))

    def blk_body(b, carry):
        row0 = s * 160 + b * 8
        pltpu.sync_copy(src_hbm.at[pl.ds(row0, 8)], sidx)
        pltpu.sync_copy(dst_hbm.at[pl.ds(row0, 8)], didx)
        pltpu.sync_copy(repl_hbm.at[pl.ds(row0, 8)], ridx)
        for r in range(8):
            for cb in range(8):
                sl = pl.ds(cb * 16, 16)
                didx[r, sl] = to_local(didx[r, sl])
                ridx[r, sl] = to_local(ridx[r, sl])
        gA = fire_g(0, rowA, gsA)
        for r in range(0, 8, 2):
            gA.wait()
            sA = fire_s(r, rowA, ssA)
            gB = fire_g(r + 1, rowB, gsB)
            gB.wait()
            sA[0].wait()
            sA[1].wait()
            sB = fire_s(r + 1, rowB, ssB)
            if r + 2 < 8:
                gA = fire_g(r + 2, rowA, gsA)
            sB[0].wait()
            sB[1].wait()
        return carry

    lax.fori_loop(0, 20, blk_body, 0)
    plsc.subcore_barrier()
    pltpu.sync_copy(acc.at[pl.ds(s * 624, 624)],
                    out_hbm.at[c].at[pl.ds(s * 624, 624)])

    @pl.when(s == 15)
    def _():
        pltpu.sync_copy(acc.at[pl.ds(9984, 16)],
                        out_hbm.at[c].at[pl.ds(9984, 16)])


_agg = pl.kernel(
    _agg_body,
    out_type=jax.ShapeDtypeStruct((2, 10000, F), jnp.float32),
    mesh=_mesh,
    scratch_types=[
        pltpu.VMEM_SHARED((10016, F), jnp.float32),
        pltpu.VMEM((8, 128), jnp.int32),
        pltpu.VMEM((8, 128), jnp.int32),
        pltpu.VMEM((8, 128), jnp.int32),
        pltpu.VMEM((128, F), jnp.float32),
        pltpu.VMEM((128, F), jnp.float32),
        pltpu.SemaphoreType.DMA,
        pltpu.SemaphoreType.DMA,
        pltpu.SemaphoreType.DMA,
        pltpu.SemaphoreType.DMA,
    ],
)


def _prologue_body(f_ref, W_ref, b_ref, t0_ref, t1_ref, t0n_ref):
    f = f_ref[...]
    t0 = jax.lax.dot_general(f, W_ref[0], (((1,), (0,)), ((), ())),
                             preferred_element_type=jnp.float32) + b_ref[0, 0]
    t1 = jax.lax.dot_general(f, W_ref[1], (((1,), (0,)), ((), ())),
                             preferred_element_type=jnp.float32) + b_ref[1, 0]
    ss = jnp.sum(t0 * t0, axis=1, keepdims=True)
    nb = jnp.maximum(jnp.sqrt(ss), 1e-8)
    t0_ref[...] = t0
    t1_ref[...] = t1
    t0n_ref[...] = t0 / nb


def _prologue(features, lin_W, lin_b):
    blk = BLK
    return pl.pallas_call(
        _prologue_body,
        grid=(N // blk,),
        in_specs=[pl.BlockSpec((blk, F), lambda i: (i, 0)),
                  pl.BlockSpec((T, F, F), lambda i: (0, 0, 0)),
                  pl.BlockSpec((T, 1, F), lambda i: (0, 0, 0))],
        out_specs=[pl.BlockSpec((blk, F), lambda i: (i, 0))] * 3,
        out_shape=[jax.ShapeDtypeStruct((N, F), jnp.float32)] * 3,
    )(features, lin_W, lin_b.reshape(T, 1, F))


BLK = 2000


def _dinv_blk(hp, i):
    base = jnp.where(i < 10, 2.0, 1.0)
    return jax.lax.rsqrt(hp[:, 0:1] + base)


def _layer_body(first, acc_ref, h_ref, hsh_ref, hp_ref, b_ref, W_ref,
                out_ref):
    i = pl.program_id(0)
    dinv = _dinv_blk(hp_ref[...], i)
    if first:
        x = jax.nn.relu(acc_ref[...])
    else:
        ex_same = jnp.where(i < 5, 1.0, 0.0)
        ex_shift = jnp.where((i >= 5) & (i < 10), 1.0, 0.0)
        pre = ((acc_ref[...] + h_ref[...] + ex_same * h_ref[...]
                + ex_shift * hsh_ref[...]) * dinv + b_ref[...])
        x = jax.nn.relu(pre)
    out_ref[...] = jax.lax.dot_general(
        x, W_ref[...], (((1,), (0,)), ((), ())),
        preferred_element_type=jnp.float32) * dinv


def _layer0(af, hpx, gcn0_W):
    blk = BLK
    dummy = jnp.zeros((8, 128), jnp.float32)
    return pl.pallas_call(
        functools.partial(_layer_body, True),
        grid=(N3 // blk,),
        in_specs=[pl.BlockSpec((blk, F), lambda i: (i, 0)),
                  pl.BlockSpec((8, 128), lambda i: (0, 0)),
                  pl.BlockSpec((8, 128), lambda i: (0, 0)),
                  pl.BlockSpec((blk, F), lambda i: (i, 0)),
                  pl.BlockSpec((8, 128), lambda i: (0, 0)),
                  pl.BlockSpec((F, F), lambda i: (0, 0))],
        out_specs=pl.BlockSpec((blk, F), lambda i: (i, 0)),
        out_shape=jax.ShapeDtypeStruct((N3, F), jnp.float32),
    )(af, dummy, dummy, hpx, dummy, gcn0_W)


def _layer_mid(accx, h, hpx, b, W):
    blk = BLK
    return pl.pallas_call(
        functools.partial(_layer_body, False),
        grid=(N3 // blk,),
        in_specs=[pl.BlockSpec((blk, F), lambda i: (i, 0)),
                  pl.BlockSpec((blk, F), lambda i: (i, 0)),
                  pl.BlockSpec((blk, F),
                               lambda i: (jnp.clip(i - 5, 0, N3 // blk - 1), 0)),
                  pl.BlockSpec((blk, F), lambda i: (i, 0)),
                  pl.BlockSpec((1, F), lambda i: (0, 0)),
                  pl.BlockSpec((F, F), lambda i: (0, 0))],
        out_specs=pl.BlockSpec((blk, F), lambda i: (i, 0)),
        out_shape=jax.ShapeDtypeStruct((N3, F), jnp.float32),
    )(accx, h, h, hpx, b.reshape(1, F), W)


def _final_body(acc_ref, h_ref, hsh_ref, hp_ref, b_ref, out_ref):
    i = pl.program_id(0)
    dinv = _dinv_blk(hp_ref[...], i)
    ex_same = jnp.where(i < 5, 1.0, 0.0)
    ex_shift = jnp.where((i >= 5) & (i < 10), 1.0, 0.0)
    out_ref[...] = ((acc_ref[...] + h_ref[...] + ex_same * h_ref[...]
                     + ex_shift * hsh_ref[...]) * dinv + b_ref[...])


def _layer_final(accx, h, hpx, b):
    blk = BLK
    return pl.pallas_call(
        _final_body,
        grid=(N3 // blk,),
        in_specs=[pl.BlockSpec((blk, F), lambda i: (i, 0)),
                  pl.BlockSpec((blk, F), lambda i: (i, 0)),
                  pl.BlockSpec((blk, F),
                               lambda i: (jnp.clip(i - 5, 0, N3 // blk - 1), 0)),
                  pl.BlockSpec((blk, F), lambda i: (i, 0)),
                  pl.BlockSpec((1, F), lambda i: (0, 0))],
        out_specs=pl.BlockSpec((blk, F), lambda i: (i, 0)),
        out_shape=jax.ShapeDtypeStruct((N3, F), jnp.float32),
    )(accx, h, h, hpx, b.reshape(1, F))


def _epilogue_a_body(af2_ref, lin1W_ref, lin1b_ref, Hs_ref, hf_ref):
    af2 = af2_ref[...]
    allv = jax.lax.dot_general(
        jax.nn.relu(af2), lin1W_ref[...], (((1,), (0,)), ((), ())),
        preferred_element_type=jnp.float32) + lin1b_ref[...]
    iota = jax.lax.broadcasted_iota(jnp.int32, (N, NUM_HE), 1)
    H = jnp.zeros((N, NUM_HE), jnp.float32)
    for i in range(T + 1):
        blk = allv[i * N:(i + 1) * N]
        m = jnp.max(blk, axis=1, keepdims=True)
        first = jnp.min(jnp.where(blk == m, iota, NUM_HE), axis=1, keepdims=True)
        H = H + jnp.where(iota == first, 1.0, 0.0)
    mask = jnp.where(H > 0, 1.0, 0.0)
    hf = jax.lax.dot_general(mask, af2[:N], (((0,), (0,)), ((), ())),
                             preferred_element_type=jnp.float32)
    e = jnp.exp(H - jnp.max(H, axis=0, keepdims=True))
    Hs_ref[...] = e / jnp.sum(e, axis=0, keepdims=True)
    hf_ref[...] = hf


def _epilogue_a(af2, lin1_W, lin1_b):
    return pl.pallas_call(
        _epilogue_a_body,
        out_shape=(jax.ShapeDtypeStruct((N, NUM_HE), jnp.float32),
                   jax.ShapeDtypeStruct((NUM_HE, F), jnp.float32)),
    )(af2, lin1_W, lin1_b.reshape(1, NUM_HE))


def _dots_body(af_ref, hf_ref, out_ref):
    out_ref[...] = jax.lax.dot_general(
        af_ref[...], hf_ref[...], (((1,), (1,)), ((), ())),
        preferred_element_type=jnp.float32) * SCALE


def _dots(all_features, hf):
    blk = 6000
    return pl.pallas_call(
        _dots_body,
        grid=(N3 // blk,),
        in_specs=[pl.BlockSpec((blk, F), lambda i: (i, 0)),
                  pl.BlockSpec((NUM_HE, F), lambda i: (0, 0))],
        out_specs=pl.BlockSpec((blk, NUM_HE), lambda i: (i, 0)),
        out_shape=jax.ShapeDtypeStruct((N3, NUM_HE), jnp.float32),
    )(all_features, hf)


def _unshuffle(part):
    return jnp.concatenate(
        [part[0, :5000], part[1, :5000], part[0, 5000:], part[1, 5000:],
         jnp.zeros((N, F), jnp.float32)], axis=0)


_SC_SELECT = True
_SC_HIST = True
_SC_AGG = True


def _select_xla(features, t0n, src, dst):
    j = (2 * jnp.arange(E)) % E
    a = features[src]
    s0 = jnp.sum(a * t0n[dst[j]], axis=1)
    s1 = jnp.sum(a * t0n[dst[j + 1]], axis=1)
    return dst + jnp.where(s1 > s0, N, 0).astype(jnp.int32)


def _hist_xla(dst, repl):
    cnt = (jnp.zeros((20000,), jnp.float32).at[dst].add(1.0)
           .at[repl].add(1.0))
    cntf = jnp.tile(cnt[:, None], (1, F))
    loc = jnp.concatenate([cntf[:5000], cntf[10000:15000],
                           cntf[5000:10000], cntf[15000:]], axis=0)
    return loc.reshape(2, 10000, F)


def _agg_xla(hs, src, dst, repl):
    rows = hs[src]
    accf = (jnp.zeros((20000, F), jnp.float32).at[dst].add(rows)
            .at[repl].add(rows))
    loc = jnp.concatenate([accf[:5000], accf[10000:15000],
                           accf[5000:10000], accf[15000:]], axis=0)
    return loc.reshape(2, 10000, F)


def kernel(edge_index, features, args, lin_W, lin_b, gcn0_W, gcn0_b, gcn1_W,
           gcn1_b, lin1_W, lin1_b):
    src = edge_index[0]
    dst = edge_index[1]

    T0, T1, t0n = _prologue(features, lin_W, lin_b)
    all_features = jnp.concatenate([features, T0, T1], axis=0)

    if _SC_SELECT:
        repl = _select(features, t0n, src, dst)
    else:
        repl = _select_xla(features, t0n, src, dst)

    padn = 2560 * 128 - E
    src_p = jnp.concatenate([src, jnp.zeros((padn,), jnp.int32)]).reshape(2560, 128)
    dst_p = jnp.concatenate([dst, jnp.full((padn,), 20000, jnp.int32)]).reshape(2560, 128)
    repl_p = jnp.concatenate([repl, jnp.full((padn,), 20000, jnp.int32)]).reshape(2560, 128)

    hpart = _hist(dst_p, repl_p) if _SC_HIST else _hist_xla(dst, repl)
    hpx = _unshuffle(hpart)

    h0s = _layer0(all_features, hpx, gcn0_W)
    part0 = (_agg(h0s, src_p, dst_p, repl_p) if _SC_AGG
             else _agg_xla(h0s, src, dst, repl))

    h1s = _layer_mid(_unshuffle(part0), h0s, hpx, gcn0_b, gcn1_W)
    part1 = (_agg(h1s, src_p, dst_p, repl_p) if _SC_AGG
             else _agg_xla(h1s, src, dst, repl))

    af2 = _layer_final(_unshuffle(part1), h1s, hpx, gcn1_b)

    Hs, hf = _epilogue_a(af2, lin1_W, lin1_b)
    dots = _dots(all_features, hf)
    return (Hs, hf, dots)
